# baseline (speedup 1.0000x reference)
import jax
import jax.numpy as jnp
from jax import lax
from jax.experimental import pallas as pl
from jax.experimental.pallas import tpu as pltpu
from jax.experimental.pallas import tpu_sc as plsc

N, C, H, W = 4, 96, 384, 384
HG, WG = 384, 384
P = HG * WG
NP = N * P
NWORK = 32
PTS_PER_W = NP // NWORK
CHUNK = 128
NCH = PTS_PER_W // CHUNK
L = 16
GPC = CHUNK // L
NB = C // 32
WPR = 64


def _sc_body(xt, gx, gy, out,
             gxv, gyv,
             i00, i01, i10, i11,
             w00, w01, w10, w11,
             r00, r01, r10, r11,
             outv,
             gsem0, gsem1, rsem0, rsem1, osem0, osem1):
    wid = lax.axis_index("s") * 2 + lax.axis_index("c")
    base0 = wid * PTS_PER_W
    rowoff = lax.shift_right_logical(wid, 3) * (H * W)
    gsems = (gsem0, gsem1)
    rsems = (rsem0, rsem1)
    osems = (osem0, osem1)

    def idxw(s):
        for g in range(GPC):
            sl = pl.ds(g * L, L)
            gxg = gxv[s, sl]
            gyg = gyv[s, sl]
            ix = gxg * (W * 0.5) + (W - 1.0) * 0.5
            iy = gyg * (H * 0.5) + (H - 1.0) * 0.5
            ix0t = ix.astype(jnp.int32)
            ix0 = jnp.where(ix0t.astype(jnp.float32) > ix, ix0t - 1, ix0t)
            iy0t = iy.astype(jnp.int32)
            iy0 = jnp.where(iy0t.astype(jnp.float32) > iy, iy0t - 1, iy0t)
            fx = ix - ix0.astype(jnp.float32)
            fy = iy - iy0.astype(jnp.float32)
            ix1 = ix0 + 1
            iy1 = iy0 + 1
            wx0 = jnp.where((ix0 >= 0) & (ix0 <= W - 1), 1.0 - fx, 0.0)
            wx1 = jnp.where((ix1 >= 0) & (ix1 <= W - 1), fx, 0.0)
            wy0 = jnp.where((iy0 >= 0) & (iy0 <= H - 1), 1.0 - fy, 0.0)
            wy1 = jnp.where((iy1 >= 0) & (iy1 <= H - 1), fy, 0.0)
            ix0c = jnp.minimum(jnp.maximum(ix0, 0), W - 1)
            ix1c = jnp.minimum(jnp.maximum(ix1, 0), W - 1)
            iy0c = jnp.minimum(jnp.maximum(iy0, 0), H - 1)
            iy1c = jnp.minimum(jnp.maximum(iy1, 0), H - 1)
            r0 = rowoff + iy0c * W
            r1 = rowoff + iy1c * W
            i00[s, sl] = r0 + ix0c
            i01[s, sl] = r0 + ix1c
            i10[s, sl] = r1 + ix0c
            i11[s, sl] = r1 + ix1c
            w00[s, sl] = wy0 * wx0
            w01[s, sl] = wy0 * wx1
            w10[s, sl] = wy1 * wx0
            w11[s, sl] = wy1 * wx1

    def gather_copies(s):
        sem = rsems[s]
        return (
            pltpu.make_async_copy(xt.at[i00.at[s]], r00.at[s], sem),
            pltpu.make_async_copy(xt.at[i01.at[s]], r01.at[s], sem),
            pltpu.make_async_copy(xt.at[i10.at[s]], r10.at[s], sem),
            pltpu.make_async_copy(xt.at[i11.at[s]], r11.at[s], sem),
        )

    def grid_copies(s, base):
        sem = gsems[s]
        return (
            pltpu.make_async_copy(gx.at[pl.ds(base, CHUNK)], gxv.at[s], sem),
            pltpu.make_async_copy(gy.at[pl.ds(base, CHUNK)], gyv.at[s], sem),
        )

    def store_copy(s, base):
        return pltpu.make_async_copy(
            outv.at[s], out.at[pl.ds(base, CHUNK)], osems[s])

    def lohi(rref, s, p, b):
        packed = rref[s, p, pl.ds(b * L, L)]
        lo = plsc.bitcast(lax.shift_left(packed, 16), jnp.float32)
        hi = plsc.bitcast(packed & jnp.int32(-65536), jnp.float32)
        return lo, hi

    def blend(s):
        ev = lax.iota(jnp.int32, L) * 2
        sv = jnp.full((L,), s, jnp.int32)

        def gbody(g, c2):
            go = g * L
            w00g = w00[s, pl.ds(go, L)]
            w01g = w01[s, pl.ds(go, L)]
            w10g = w10[s, pl.ds(go, L)]
            w11g = w11[s, pl.ds(go, L)]
            for k in range(L):
                p = go + k
                pv = jnp.full((L,), p, jnp.int32)
                s00 = jnp.full((L,), w00g[k])
                s01 = jnp.full((L,), w01g[k])
                s10 = jnp.full((L,), w10g[k])
                s11 = jnp.full((L,), w11g[k])
                for b in range(NB):
                    a00, b00 = lohi(r00, s, p, b)
                    a01, b01 = lohi(r01, s, p, b)
                    a10, b10 = lohi(r10, s, p, b)
                    a11, b11 = lohi(r11, s, p, b)
                    acc_lo = (a00 * s00 + a01 * s01
                              + a10 * s10 + a11 * s11)
                    acc_hi = (b00 * s00 + b01 * s01
                              + b10 * s10 + b11 * s11)
                    plsc.store_scatter(outv, [sv, pv, ev + (32 * b)],
                                       acc_lo)
                    plsc.store_scatter(outv, [sv, pv, ev + (32 * b + 1)],
                                       acc_hi)
            return c2

        lax.fori_loop(0, GPC, gbody, 0)

    def proc(i, s):
        q = 1 - s

        @pl.when(i + 1 < NCH)
        def _():
            for cp in grid_copies(q, 0):
                cp.wait()
            idxw(q)
            for cp in gather_copies(q):
                cp.start()

        @pl.when(i + 2 < NCH)
        def _():
            for cp in grid_copies(s, base0 + (i + 2) * CHUNK):
                cp.start()

        for cp in gather_copies(s):
            cp.wait()

        @pl.when(i >= 2)
        def _():
            store_copy(s, 0).wait()

        blend(s)
        store_copy(s, base0 + i * CHUNK).start()

    for cp in grid_copies(0, base0):
        cp.start()
    for cp in grid_copies(0, base0):
        cp.wait()
    idxw(0)
    for cp in gather_copies(0):
        cp.start()
    for cp in grid_copies(1, base0 + CHUNK):
        cp.start()

    def pair(sidx, c2):
        proc(sidx * 2, 0)
        proc(sidx * 2 + 1, 1)
        return c2

    lax.fori_loop(0, NCH // 2, pair, 0)

    store_copy(0, 0).wait()
    store_copy(1, 0).wait()


@jax.jit
def kernel(x, grid):
    xt = jnp.transpose(x, (0, 2, 3, 1)).reshape(N * H * W, C)
    xt = lax.optimization_barrier(xt)
    xtp = lax.bitcast_convert_type(
        xt.astype(jnp.bfloat16).reshape(N * H * W, C // 2, 2), jnp.int32)
    xtp = jnp.pad(xtp, ((0, 0), (0, WPR - C // 2)))
    gx = grid[..., 0].reshape(NP)
    gy = grid[..., 1].reshape(NP)

    mesh = plsc.VectorSubcoreMesh(core_axis_name="c", subcore_axis_name="s")
    run = pl.kernel(
        _sc_body,
        mesh=mesh,
        compiler_params=pltpu.CompilerParams(
            needs_layout_passes=False,
            use_tc_tiling_on_sc=False,
        ),
        out_type=jax.ShapeDtypeStruct((NP, C), jnp.float32),
        scratch_types=[
            pltpu.VMEM((2, CHUNK), jnp.float32),
            pltpu.VMEM((2, CHUNK), jnp.float32),
            pltpu.VMEM((2, CHUNK), jnp.int32),
            pltpu.VMEM((2, CHUNK), jnp.int32),
            pltpu.VMEM((2, CHUNK), jnp.int32),
            pltpu.VMEM((2, CHUNK), jnp.int32),
            pltpu.VMEM((2, CHUNK), jnp.float32),
            pltpu.VMEM((2, CHUNK), jnp.float32),
            pltpu.VMEM((2, CHUNK), jnp.float32),
            pltpu.VMEM((2, CHUNK), jnp.float32),
            pltpu.VMEM((2, CHUNK, WPR), jnp.int32),
            pltpu.VMEM((2, CHUNK, WPR), jnp.int32),
            pltpu.VMEM((2, CHUNK, WPR), jnp.int32),
            pltpu.VMEM((2, CHUNK, WPR), jnp.int32),
            pltpu.VMEM((2, CHUNK, C), jnp.float32),
            pltpu.SemaphoreType.DMA,
            pltpu.SemaphoreType.DMA,
            pltpu.SemaphoreType.DMA,
            pltpu.SemaphoreType.DMA,
            pltpu.SemaphoreType.DMA,
            pltpu.SemaphoreType.DMA,
        ],
    )
    yt = run(xtp, gx, gy)
    return yt.reshape(N, HG, WG, C).transpose(0, 3, 1, 2)


# speedup vs baseline: 1.9922x; 1.9922x over previous
import jax
import jax.numpy as jnp
from jax import lax
from jax.experimental import pallas as pl
from jax.experimental.pallas import tpu as pltpu
from jax.experimental.pallas import tpu_sc as plsc

N, C, H, W = 4, 96, 384, 384
HG, WG = 384, 384
P = HG * WG
NP = N * P
NWORK = 32
PTS_PER_W = P // NWORK
CHUNK = 128
NCH = PTS_PER_W // CHUNK
L = 16
GPC = CHUNK // L
NB = C // 32
WPR = 48


def _sc_body(xt, gx, gy, out,
             gxv, gyv,
             i00, i01, i10, i11,
             w00, w01, w10, w11,
             r00, r01, r10, r11,
             outv,
             gsem0, gsem1, rsem0, rsem1, osem0, osem1):
    wid = lax.axis_index("s") * 2 + lax.axis_index("c")
    base0 = wid * PTS_PER_W
    rowoff = 0
    gsems = (gsem0, gsem1)
    rsems = (rsem0, rsem1)
    osems = (osem0, osem1)

    def idxw(s):
        for g in range(GPC):
            sl = pl.ds(g * L, L)
            gxg = gxv[s, sl]
            gyg = gyv[s, sl]
            ix = gxg * (W * 0.5) + (W - 1.0) * 0.5
            iy = gyg * (H * 0.5) + (H - 1.0) * 0.5
            ix0t = ix.astype(jnp.int32)
            ix0 = jnp.where(ix0t.astype(jnp.float32) > ix, ix0t - 1, ix0t)
            iy0t = iy.astype(jnp.int32)
            iy0 = jnp.where(iy0t.astype(jnp.float32) > iy, iy0t - 1, iy0t)
            fx = ix - ix0.astype(jnp.float32)
            fy = iy - iy0.astype(jnp.float32)
            ix1 = ix0 + 1
            iy1 = iy0 + 1
            wx0 = jnp.where((ix0 >= 0) & (ix0 <= W - 1), 1.0 - fx, 0.0)
            wx1 = jnp.where((ix1 >= 0) & (ix1 <= W - 1), fx, 0.0)
            wy0 = jnp.where((iy0 >= 0) & (iy0 <= H - 1), 1.0 - fy, 0.0)
            wy1 = jnp.where((iy1 >= 0) & (iy1 <= H - 1), fy, 0.0)
            ix0c = jnp.minimum(jnp.maximum(ix0, 0), W - 1)
            ix1c = jnp.minimum(jnp.maximum(ix1, 0), W - 1)
            iy0c = jnp.minimum(jnp.maximum(iy0, 0), H - 1)
            iy1c = jnp.minimum(jnp.maximum(iy1, 0), H - 1)
            r0 = rowoff + iy0c * W
            r1 = rowoff + iy1c * W
            i00[s, sl] = r0 + ix0c
            i01[s, sl] = r0 + ix1c
            i10[s, sl] = r1 + ix0c
            i11[s, sl] = r1 + ix1c
            w00[s, sl] = wy0 * wx0
            w01[s, sl] = wy0 * wx1
            w10[s, sl] = wy1 * wx0
            w11[s, sl] = wy1 * wx1

    def gather_copies(s):
        sem = rsems[s]
        return (
            pltpu.make_async_copy(xt.at[i00.at[s]], r00.at[s], sem),
            pltpu.make_async_copy(xt.at[i01.at[s]], r01.at[s], sem),
            pltpu.make_async_copy(xt.at[i10.at[s]], r10.at[s], sem),
            pltpu.make_async_copy(xt.at[i11.at[s]], r11.at[s], sem),
        )

    def grid_copies(s, base):
        sem = gsems[s]
        return (
            pltpu.make_async_copy(gx.at[pl.ds(base, CHUNK)], gxv.at[s], sem),
            pltpu.make_async_copy(gy.at[pl.ds(base, CHUNK)], gyv.at[s], sem),
        )

    def store_copy(s, base):
        return pltpu.make_async_copy(
            outv.at[s], out.at[pl.ds(base, CHUNK)], osems[s])

    def lohi(rref, s, p, b):
        packed = rref[s, p, pl.ds(b * L, L)]
        lo = plsc.bitcast(lax.shift_left(packed, jnp.uint32(16)), jnp.float32)
        hi = plsc.bitcast(packed, jnp.float32)
        return lo, hi

    def blend(s):
        def gbody(g, c2):
            go = g * L
            w00g = w00[s, pl.ds(go, L)]
            w01g = w01[s, pl.ds(go, L)]
            w10g = w10[s, pl.ds(go, L)]
            w11g = w11[s, pl.ds(go, L)]
            for k in range(L):
                p = go + k
                s00 = jnp.full((L,), w00g[k])
                s01 = jnp.full((L,), w01g[k])
                s10 = jnp.full((L,), w10g[k])
                s11 = jnp.full((L,), w11g[k])
                for b in range(NB):
                    a00, b00 = lohi(r00, s, p, b)
                    a01, b01 = lohi(r01, s, p, b)
                    a10, b10 = lohi(r10, s, p, b)
                    a11, b11 = lohi(r11, s, p, b)
                    acc_lo = (a00 * s00 + a01 * s01
                              + a10 * s10 + a11 * s11)
                    acc_hi = (b00 * s00 + b01 * s01
                              + b10 * s10 + b11 * s11)
                    outv[s, p, pl.ds(b * L, L)] = acc_lo
                    outv[s, p, pl.ds(C // 2 + b * L, L)] = acc_hi
            return c2

        lax.fori_loop(0, GPC, gbody, 0)

    def proc(i, s):
        q = 1 - s

        @pl.when(i + 1 < NCH)
        def _():
            for cp in grid_copies(q, 0):
                cp.wait()
            idxw(q)
            for cp in gather_copies(q):
                cp.start()

        @pl.when(i + 2 < NCH)
        def _():
            for cp in grid_copies(s, base0 + (i + 2) * CHUNK):
                cp.start()

        for cp in gather_copies(s):
            cp.wait()

        @pl.when(i >= 2)
        def _():
            store_copy(s, 0).wait()

        blend(s)
        store_copy(s, base0 + i * CHUNK).start()

    for cp in grid_copies(0, base0):
        cp.start()
    for cp in grid_copies(0, base0):
        cp.wait()
    idxw(0)
    for cp in gather_copies(0):
        cp.start()
    for cp in grid_copies(1, base0 + CHUNK):
        cp.start()

    def pair(sidx, c2):
        proc(sidx * 2, 0)
        proc(sidx * 2 + 1, 1)
        return c2

    lax.fori_loop(0, NCH // 2, pair, 0)

    store_copy(0, 0).wait()
    store_copy(1, 0).wait()


@jax.jit
def kernel(x, grid):
    mesh = plsc.VectorSubcoreMesh(core_axis_name="c", subcore_axis_name="s")
    run = pl.kernel(
        _sc_body,
        mesh=mesh,
        compiler_params=pltpu.CompilerParams(
            needs_layout_passes=False,
            use_tc_tiling_on_sc=False,
        ),
        out_type=jax.ShapeDtypeStruct((P, C), jnp.float32),
        scratch_types=[
            pltpu.VMEM((2, CHUNK), jnp.float32),
            pltpu.VMEM((2, CHUNK), jnp.float32),
            pltpu.VMEM((2, CHUNK), jnp.int32),
            pltpu.VMEM((2, CHUNK), jnp.int32),
            pltpu.VMEM((2, CHUNK), jnp.int32),
            pltpu.VMEM((2, CHUNK), jnp.int32),
            pltpu.VMEM((2, CHUNK), jnp.float32),
            pltpu.VMEM((2, CHUNK), jnp.float32),
            pltpu.VMEM((2, CHUNK), jnp.float32),
            pltpu.VMEM((2, CHUNK), jnp.float32),
            pltpu.VMEM((2, CHUNK, WPR), jnp.uint32),
            pltpu.VMEM((2, CHUNK, WPR), jnp.uint32),
            pltpu.VMEM((2, CHUNK, WPR), jnp.uint32),
            pltpu.VMEM((2, CHUNK, WPR), jnp.uint32),
            pltpu.VMEM((2, CHUNK, C), jnp.float32),
            pltpu.SemaphoreType.DMA,
            pltpu.SemaphoreType.DMA,
            pltpu.SemaphoreType.DMA,
            pltpu.SemaphoreType.DMA,
            pltpu.SemaphoreType.DMA,
            pltpu.SemaphoreType.DMA,
        ],
    )

    parts = []
    for n in range(N):
        au = lax.bitcast_convert_type(
            x[n, :C // 2].astype(jnp.bfloat16).astype(jnp.float32),
            jnp.uint32)
        bu = lax.bitcast_convert_type(
            x[n, C // 2:].astype(jnp.bfloat16).astype(jnp.float32),
            jnp.uint32)
        word = (au >> 16) | (bu & jnp.uint32(0xFFFF0000))
        word = lax.optimization_barrier(word)
        xtp = jnp.transpose(word, (1, 2, 0)).reshape(P, WPR)
        gx = grid[n, ..., 0].reshape(P)
        gy = grid[n, ..., 1].reshape(P)
        yt = run(xtp, gx, gy)
        parts.append(yt.reshape(HG, WG, C).transpose(2, 0, 1))
    out = jnp.empty((N, C, HG, WG), jnp.float32)
    for n in range(N):
        out = lax.dynamic_update_slice(out, parts[n][None], (n, 0, 0, 0))
    return out
